# baseline (speedup 1.0000x reference)
import functools

import jax
import jax.numpy as jnp
from jax import lax
from jax.experimental import pallas as pl
from jax.experimental.pallas import tpu as pltpu
from jax.experimental.pallas import tpu_sc as plsc

N = 10000
E = 320000
C = 128
K = 27
EPS = 1e-5

NC = 2
NS = 16
NW = NC * NS
EW = E // NW
CHUNK = 80
NCHUNK = EW // CHUNK
SPLIT = 63
STRIPE = 632
LAST = N - 15 * STRIPE

_mesh = plsc.VectorSubcoreMesh(core_axis_name="c", subcore_axis_name="s")


@functools.partial(
    pl.kernel,
    mesh=_mesh,
    out_type=jax.ShapeDtypeStruct((NC * 2, N, C), jnp.bfloat16),
    scratch_types=[
        pltpu.VMEM((NCHUNK, CHUNK), jnp.int32),
        pltpu.VMEM((NCHUNK, CHUNK), jnp.int32),
        pltpu.VMEM((CHUNK, C), jnp.bfloat16),
        pltpu.VMEM((CHUNK, C), jnp.bfloat16),
        pltpu.VMEM((8, C), jnp.bfloat16),
        pltpu.SemaphoreType.DMA,
        pltpu.SemaphoreType.DMA,
        pltpu.VMEM_SHARED((N, C), jnp.bfloat16),
        pltpu.VMEM_SHARED((N, C), jnp.bfloat16),
    ],
    compiler_params=pltpu.CompilerParams(use_tc_tiling_on_sc=False),
)
def _edge_accum(y_hbm, g_hbm, d_hbm, out_hbm,
                gi_v, di_v, rows_a, rows_b, zb, sem_a, sem_b, acc_a, acc_b):
    cid = lax.axis_index("c")
    sid = lax.axis_index("s")
    wid = sid * NC + cid

    pltpu.sync_copy(g_hbm.at[wid], gi_v)
    pltpu.sync_copy(d_hbm.at[wid], di_v)

    zv = jnp.zeros((32,), jnp.bfloat16)
    for i in range(8):
        def zrow(j, _, i=i):
            zb[i, pl.ds(j * 32, 32)] = zv
            return 0
        lax.fori_loop(0, C // 32, zrow, 0)

    for acc in (acc_a, acc_b):
        @pl.when(sid < NS - 1)
        def _(acc=acc):
            def zcp(r, _):
                pltpu.sync_copy(zb, acc.at[pl.ds(sid * STRIPE + r * 8, 8)])
                return 0
            lax.fori_loop(0, STRIPE // 8, zcp, 0)

        @pl.when(sid == NS - 1)
        def _(acc=acc):
            def zcp(r, _):
                pltpu.sync_copy(zb, acc.at[pl.ds((NS - 1) * STRIPE + r * 8, 8)])
                return 0
            lax.fori_loop(0, LAST // 8, zcp, 0)

    plsc.subcore_barrier()

    pltpu.async_copy(y_hbm.at[gi_v.at[0]], rows_a, sem_a)

    def step(j, _):
        def issue(nbuf, nsem):
            pltpu.async_copy(y_hbm.at[gi_v.at[j + 1]], nbuf, nsem)

        def drain(buf, sem):
            pltpu.make_async_copy(y_hbm.at[gi_v.at[0]], buf, sem).wait()

            @pl.when(j < SPLIT)
            def _():
                pltpu.sync_copy(buf, acc_a.at[di_v.at[j]], add=True)

            @pl.when(j >= SPLIT)
            def _():
                pltpu.sync_copy(buf, acc_b.at[di_v.at[j]], add=True)

        @pl.when(j % 2 == 0)
        def _():
            @pl.when(j + 1 < NCHUNK)
            def _():
                issue(rows_b, sem_b)
            drain(rows_a, sem_a)

        @pl.when(j % 2 == 1)
        def _():
            @pl.when(j + 1 < NCHUNK)
            def _():
                issue(rows_a, sem_a)
            drain(rows_b, sem_b)

        return 0

    lax.fori_loop(0, NCHUNK, step, 0)
    plsc.subcore_barrier()

    for a, acc in enumerate((acc_a, acc_b)):
        @pl.when(sid < NS - 1)
        def _(a=a, acc=acc):
            pltpu.sync_copy(acc.at[pl.ds(sid * STRIPE, STRIPE)],
                            out_hbm.at[cid * 2 + a].at[pl.ds(sid * STRIPE, STRIPE)])

        @pl.when(sid == NS - 1)
        def _(a=a, acc=acc):
            pltpu.sync_copy(acc.at[pl.ds((NS - 1) * STRIPE, LAST)],
                            out_hbm.at[cid * 2 + a].at[pl.ds((NS - 1) * STRIPE, LAST)])


def _mm_body(x_ref, w_ref, y_ref):
    y = jnp.dot(x_ref[...], w_ref[0], preferred_element_type=jnp.float32)
    y_ref[0] = y.astype(jnp.bfloat16)


def _mm(xin, W):
    return pl.pallas_call(
        _mm_body,
        grid=(K,),
        in_specs=[
            pl.BlockSpec((N, C), lambda k: (0, 0)),
            pl.BlockSpec((1, C, C), lambda k: (k, 0, 0)),
        ],
        out_specs=pl.BlockSpec((1, N, C), lambda k: (k, 0, 0)),
        out_shape=jax.ShapeDtypeStruct((K, N, C), jnp.bfloat16),
    )(xin, W)


def _bn_from(p_ref, g_ref, b_ref):
    h = (p_ref[0].astype(jnp.float32) + p_ref[1].astype(jnp.float32)
         + p_ref[2].astype(jnp.float32) + p_ref[3].astype(jnp.float32))
    mu = jnp.mean(h, axis=0, keepdims=True)
    var = jnp.mean((h - mu) * (h - mu), axis=0, keepdims=True)
    return g_ref[...] * (h - mu) * lax.rsqrt(var + EPS) + b_ref[...]


def _mid_body(p_ref, g_ref, b_ref, w_ref, y_ref, h_scr):
    @pl.when(pl.program_id(0) == 0)
    def _():
        h_scr[...] = jnp.maximum(_bn_from(p_ref, g_ref, b_ref), 0.0)

    y = jnp.dot(h_scr[...], w_ref[0], preferred_element_type=jnp.float32)
    y_ref[0] = y.astype(jnp.bfloat16)


def _mid(p, g, b, W):
    return pl.pallas_call(
        _mid_body,
        grid=(K,),
        in_specs=[
            pl.BlockSpec((NC * 2, N, C), lambda k: (0, 0, 0)),
            pl.BlockSpec((1, C), lambda k: (0, 0)),
            pl.BlockSpec((1, C), lambda k: (0, 0)),
            pl.BlockSpec((1, C, C), lambda k: (k, 0, 0)),
        ],
        out_specs=pl.BlockSpec((1, N, C), lambda k: (k, 0, 0)),
        out_shape=jax.ShapeDtypeStruct((K, N, C), jnp.bfloat16),
        scratch_shapes=[pltpu.VMEM((N, C), jnp.float32)],
    )(p, g, b, W)


def _final_body(p_ref, g_ref, b_ref, x_ref, o_ref):
    o_ref[...] = jnp.maximum(_bn_from(p_ref, g_ref, b_ref) + x_ref[...], 0.0)


def _final(p, g, b, x):
    return pl.pallas_call(
        _final_body,
        grid=(1,),
        in_specs=[
            pl.BlockSpec((NC * 2, N, C), lambda k: (0, 0, 0)),
            pl.BlockSpec((1, C), lambda k: (0, 0)),
            pl.BlockSpec((1, C), lambda k: (0, 0)),
            pl.BlockSpec((N, C), lambda k: (0, 0)),
        ],
        out_specs=pl.BlockSpec((N, C), lambda k: (0, 0)),
        out_shape=jax.ShapeDtypeStruct((N, C), jnp.float32),
    )(p, g, b, x)


def kernel(x, W1, g1, b1, W2, g2, b2, edge_index, edge_offset):
    src = edge_index[0]
    dst = edge_index[1]
    gidx = (edge_offset * N + src).reshape(NW, NCHUNK, CHUNK)
    didx = dst.reshape(NW, NCHUNK, CHUNK)
    g1r = g1.reshape(1, C)
    b1r = b1.reshape(1, C)
    g2r = g2.reshape(1, C)
    b2r = b2.reshape(1, C)

    y1 = _mm(x, W1).reshape(K * N, C)
    p1 = _edge_accum(y1, gidx, didx)
    y2 = _mid(p1, g1r, b1r, W2).reshape(K * N, C)
    p2 = _edge_accum(y2, gidx, didx)
    return _final(p2, g2r, b2r, x)


# speedup vs baseline: 2.2177x; 2.2177x over previous
import functools

import jax
import jax.numpy as jnp
from jax import lax
from jax.experimental import pallas as pl
from jax.experimental.pallas import tpu as pltpu
from jax.experimental.pallas import tpu_sc as plsc

N = 10000
E = 320000
C = 128
K = 27
EPS = 1e-5

NC = 2
NS = 16
NW = NC * NS
EW = E // NW
CHUNK = 80
NCHUNK = EW // CHUNK
NBUF = 4
STRIPE = 632
LAST = N - 15 * STRIPE

_mesh = plsc.VectorSubcoreMesh(core_axis_name="c", subcore_axis_name="s")


@functools.partial(
    pl.kernel,
    mesh=_mesh,
    out_type=jax.ShapeDtypeStruct((NC, N, C), jnp.float32),
    scratch_types=[
        pltpu.VMEM((NCHUNK, CHUNK), jnp.int32),
        pltpu.VMEM((NCHUNK, CHUNK), jnp.int32),
        pltpu.VMEM((CHUNK, C), jnp.float32),
        pltpu.VMEM((CHUNK, C), jnp.float32),
        pltpu.VMEM((CHUNK, C), jnp.float32),
        pltpu.VMEM((CHUNK, C), jnp.float32),
        pltpu.VMEM((8, C), jnp.float32),
        pltpu.SemaphoreType.DMA,
        pltpu.SemaphoreType.DMA,
        pltpu.SemaphoreType.DMA,
        pltpu.SemaphoreType.DMA,
        pltpu.VMEM_SHARED((N, C), jnp.float32),
    ],
    compiler_params=pltpu.CompilerParams(use_tc_tiling_on_sc=False),
)
def _edge_accum(y_hbm, g_hbm, d_hbm, out_hbm,
                gi_v, di_v, rows0, rows1, rows2, rows3, zb,
                gsem0, gsem1, gsem2, gsem3, acc):
    rows = (rows0, rows1, rows2, rows3)
    gsem = (gsem0, gsem1, gsem2, gsem3)
    cid = lax.axis_index("c")
    sid = lax.axis_index("s")
    wid = sid * NC + cid

    pltpu.sync_copy(g_hbm.at[wid], gi_v)
    pltpu.sync_copy(d_hbm.at[wid], di_v)

    zv = jnp.zeros((16,), jnp.float32)
    for i in range(8):
        def zrow(j, _, i=i):
            zb[i, pl.ds(j * 16, 16)] = zv
            return 0
        lax.fori_loop(0, C // 16, zrow, 0)

    @pl.when(sid < NS - 1)
    def _():
        def zcp(r, _):
            pltpu.sync_copy(zb, acc.at[pl.ds(sid * STRIPE + r * 8, 8)])
            return 0
        lax.fori_loop(0, STRIPE // 8, zcp, 0)

    @pl.when(sid == NS - 1)
    def _():
        def zcp(r, _):
            pltpu.sync_copy(zb, acc.at[pl.ds((NS - 1) * STRIPE + r * 8, 8)])
            return 0
        lax.fori_loop(0, LAST // 8, zcp, 0)

    plsc.subcore_barrier()

    def gather(j, b):
        pltpu.async_copy(y_hbm.at[gi_v.at[j]], rows[b], gsem[b])

    def gather_wait(b):
        pltpu.make_async_copy(y_hbm.at[gi_v.at[0]], rows[b], gsem[b]).wait()

    def scatter(j, b):
        pltpu.async_copy(rows[b], acc.at[di_v.at[j]], ssem[b], add=True)

    def scatter_wait(b):
        pltpu.make_async_copy(y_hbm.at[gi_v.at[0]], rows[b], ssem[b]).wait()

    gather(0, 0)

    def step(j, _):
        for b in range(2):
            @pl.when(j % 2 == b)
            def _(b=b):
                @pl.when(j + 1 < NCHUNK)
                def _():
                    gather(j + 1, 1 - b)
                gather_wait(b)
                pltpu.sync_copy(rows[b], acc.at[di_v.at[j]], add=True)
        return 0

    lax.fori_loop(0, NCHUNK, step, 0)
    plsc.subcore_barrier()

    @pl.when(sid < NS - 1)
    def _():
        pltpu.sync_copy(acc.at[pl.ds(sid * STRIPE, STRIPE)],
                        out_hbm.at[cid].at[pl.ds(sid * STRIPE, STRIPE)])

    @pl.when(sid == NS - 1)
    def _():
        pltpu.sync_copy(acc.at[pl.ds((NS - 1) * STRIPE, LAST)],
                        out_hbm.at[cid].at[pl.ds((NS - 1) * STRIPE, LAST)])


def _mm_body(x_ref, w_ref, o_ref, s_ref, y_ref, gi_ref):
    @pl.when(pl.program_id(0) == 0)
    def _():
        gi_ref[...] = o_ref[...] * N + s_ref[...]

    y_ref[0] = jnp.dot(x_ref[...], w_ref[0], preferred_element_type=jnp.float32)


def _mm(xin, W, off2, src2):
    return pl.pallas_call(
        _mm_body,
        grid=(K,),
        in_specs=[
            pl.BlockSpec((N, C), lambda k: (0, 0)),
            pl.BlockSpec((1, C, C), lambda k: (k, 0, 0)),
            pl.BlockSpec((E // CHUNK, CHUNK), lambda k: (0, 0)),
            pl.BlockSpec((E // CHUNK, CHUNK), lambda k: (0, 0)),
        ],
        out_specs=[
            pl.BlockSpec((1, N, C), lambda k: (k, 0, 0)),
            pl.BlockSpec((E // CHUNK, CHUNK), lambda k: (0, 0)),
        ],
        out_shape=[
            jax.ShapeDtypeStruct((K, N, C), jnp.float32),
            jax.ShapeDtypeStruct((E // CHUNK, CHUNK), jnp.int32),
        ],
    )(xin, W, off2, src2)


def _bn_from(p_ref, g_ref, b_ref):
    h = p_ref[0] + p_ref[1]
    mu = jnp.mean(h, axis=0, keepdims=True)
    var = jnp.mean((h - mu) * (h - mu), axis=0, keepdims=True)
    return g_ref[...] * (h - mu) * lax.rsqrt(var + EPS) + b_ref[...]


def _mid_body(p_ref, g_ref, b_ref, w_ref, y_ref, h_scr):
    @pl.when(pl.program_id(0) == 0)
    def _():
        h_scr[...] = jnp.maximum(_bn_from(p_ref, g_ref, b_ref), 0.0)

    y_ref[0] = jnp.dot(h_scr[...], w_ref[0], preferred_element_type=jnp.float32)


def _mid(p, g, b, W):
    return pl.pallas_call(
        _mid_body,
        grid=(K,),
        in_specs=[
            pl.BlockSpec((NC, N, C), lambda k: (0, 0, 0)),
            pl.BlockSpec((1, C), lambda k: (0, 0)),
            pl.BlockSpec((1, C), lambda k: (0, 0)),
            pl.BlockSpec((1, C, C), lambda k: (k, 0, 0)),
        ],
        out_specs=pl.BlockSpec((1, N, C), lambda k: (k, 0, 0)),
        out_shape=jax.ShapeDtypeStruct((K, N, C), jnp.float32),
        scratch_shapes=[pltpu.VMEM((N, C), jnp.float32)],
    )(p, g, b, W)


def _final_body(p_ref, g_ref, b_ref, x_ref, o_ref):
    o_ref[...] = jnp.maximum(_bn_from(p_ref, g_ref, b_ref) + x_ref[...], 0.0)


def _final(p, g, b, x):
    return pl.pallas_call(
        _final_body,
        grid=(1,),
        in_specs=[
            pl.BlockSpec((NC, N, C), lambda k: (0, 0, 0)),
            pl.BlockSpec((1, C), lambda k: (0, 0)),
            pl.BlockSpec((1, C), lambda k: (0, 0)),
            pl.BlockSpec((N, C), lambda k: (0, 0)),
        ],
        out_specs=pl.BlockSpec((N, C), lambda k: (0, 0)),
        out_shape=jax.ShapeDtypeStruct((N, C), jnp.float32),
    )(p, g, b, x)


def kernel(x, W1, g1, b1, W2, g2, b2, edge_index, edge_offset):
    src2 = edge_index[0].reshape(E // CHUNK, CHUNK)
    off2 = edge_offset.reshape(E // CHUNK, CHUNK)
    didx = edge_index[1].reshape(NW, NCHUNK, CHUNK)
    g1r = g1.reshape(1, C)
    b1r = b1.reshape(1, C)
    g2r = g2.reshape(1, C)
    b2r = b2.reshape(1, C)

    y1, gidx = _mm(x, W1, off2, src2)
    y1 = y1.reshape(K * N, C)
    gidx = gidx.reshape(NW, NCHUNK, CHUNK)
    p1 = _edge_accum(y1, gidx, didx)
    y2 = _mid(p1, g1r, b1r, W2).reshape(K * N, C)
    p2 = _edge_accum(y2, gidx, didx)
    return _final(p2, g2r, b2r, x)


# speedup vs baseline: 2.2742x; 1.0255x over previous
import functools

import jax
import jax.numpy as jnp
from jax import lax
from jax.experimental import pallas as pl
from jax.experimental.pallas import tpu as pltpu
from jax.experimental.pallas import tpu_sc as plsc

N = 10000
E = 320000
C = 128
K = 27
EPS = 1e-5

NC = 2
NS = 16
NW = NC * NS
EW = E // NW
CHUNK = 80
NCHUNK = EW // CHUNK
NBUF = 4
STRIPE = 632
LAST = N - 15 * STRIPE

_mesh = plsc.VectorSubcoreMesh(core_axis_name="c", subcore_axis_name="s")


@functools.partial(
    pl.kernel,
    mesh=_mesh,
    out_type=jax.ShapeDtypeStruct((NC, N, C), jnp.float32),
    scratch_types=[
        pltpu.VMEM((NCHUNK, CHUNK), jnp.int32),
        pltpu.VMEM((NCHUNK, CHUNK), jnp.int32),
        pltpu.VMEM((CHUNK, C), jnp.float32),
        pltpu.VMEM((CHUNK, C), jnp.float32),
        pltpu.VMEM((CHUNK, C), jnp.float32),
        pltpu.VMEM((CHUNK, C), jnp.float32),
        pltpu.VMEM((8, C), jnp.float32),
        pltpu.SemaphoreType.DMA,
        pltpu.SemaphoreType.DMA,
        pltpu.SemaphoreType.DMA,
        pltpu.SemaphoreType.DMA,
        pltpu.VMEM_SHARED((N, C), jnp.float32),
    ],
    compiler_params=pltpu.CompilerParams(use_tc_tiling_on_sc=False),
)
def _edge_accum(y_hbm, g_hbm, d_hbm, out_hbm,
                gi_v, di_v, rows0, rows1, rows2, rows3, zb,
                gsem0, gsem1, gsem2, gsem3, acc):
    rows = (rows0, rows1, rows2, rows3)
    gsem = (gsem0, gsem1, gsem2, gsem3)
    cid = lax.axis_index("c")
    sid = lax.axis_index("s")
    wid = sid * NC + cid

    pltpu.sync_copy(g_hbm.at[wid], gi_v)
    pltpu.sync_copy(d_hbm.at[wid], di_v)

    zv = jnp.zeros((16,), jnp.float32)
    for i in range(8):
        def zrow(j, _, i=i):
            zb[i, pl.ds(j * 16, 16)] = zv
            return 0
        lax.fori_loop(0, C // 16, zrow, 0)

    @pl.when(sid < NS - 1)
    def _():
        def zcp(r, _):
            pltpu.sync_copy(zb, acc.at[pl.ds(sid * STRIPE + r * 8, 8)])
            return 0
        lax.fori_loop(0, STRIPE // 8, zcp, 0)

    @pl.when(sid == NS - 1)
    def _():
        def zcp(r, _):
            pltpu.sync_copy(zb, acc.at[pl.ds((NS - 1) * STRIPE + r * 8, 8)])
            return 0
        lax.fori_loop(0, LAST // 8, zcp, 0)

    plsc.subcore_barrier()

    def gather(j, b):
        pltpu.async_copy(y_hbm.at[gi_v.at[j]], rows[b], gsem[b])

    def gather_wait(b):
        pltpu.make_async_copy(y_hbm.at[gi_v.at[0]], rows[b], gsem[b]).wait()

    def scatter(j, b):
        pltpu.async_copy(rows[b], acc.at[di_v.at[j]], ssem[b], add=True)

    def scatter_wait(b):
        pltpu.make_async_copy(y_hbm.at[gi_v.at[0]], rows[b], ssem[b]).wait()

    gather(0, 0)

    def step(j, _):
        for b in range(2):
            @pl.when(j % 2 == b)
            def _(b=b):
                @pl.when(j + 1 < NCHUNK)
                def _():
                    gather(j + 1, 1 - b)
                gather_wait(b)
                pltpu.sync_copy(rows[b], acc.at[di_v.at[j]], add=True)
        return 0

    lax.fori_loop(0, NCHUNK, step, 0)
    plsc.subcore_barrier()

    @pl.when(sid < NS - 1)
    def _():
        pltpu.sync_copy(acc.at[pl.ds(sid * STRIPE, STRIPE)],
                        out_hbm.at[cid].at[pl.ds(sid * STRIPE, STRIPE)])

    @pl.when(sid == NS - 1)
    def _():
        pltpu.sync_copy(acc.at[pl.ds((NS - 1) * STRIPE, LAST)],
                        out_hbm.at[cid].at[pl.ds((NS - 1) * STRIPE, LAST)])


def _mm_body(x_ref, w_ref, y_ref):
    y_ref[0] = jnp.dot(x_ref[...], w_ref[0], preferred_element_type=jnp.float32)


def _mm(xin, W):
    return pl.pallas_call(
        _mm_body,
        grid=(K,),
        in_specs=[
            pl.BlockSpec((N, C), lambda k: (0, 0)),
            pl.BlockSpec((1, C, C), lambda k: (k, 0, 0)),
        ],
        out_specs=pl.BlockSpec((1, N, C), lambda k: (k, 0, 0)),
        out_shape=jax.ShapeDtypeStruct((K, N, C), jnp.float32),
    )(xin, W)


def _bn_from(p_ref, g_ref, b_ref):
    h = p_ref[0] + p_ref[1]
    mu = jnp.mean(h, axis=0, keepdims=True)
    var = jnp.mean((h - mu) * (h - mu), axis=0, keepdims=True)
    return g_ref[...] * (h - mu) * lax.rsqrt(var + EPS) + b_ref[...]


def _mid_body(p_ref, g_ref, b_ref, w_ref, y_ref, h_scr):
    @pl.when(pl.program_id(0) == 0)
    def _():
        h_scr[...] = jnp.maximum(_bn_from(p_ref, g_ref, b_ref), 0.0)

    y_ref[0] = jnp.dot(h_scr[...], w_ref[0], preferred_element_type=jnp.float32)


def _mid(p, g, b, W):
    return pl.pallas_call(
        _mid_body,
        grid=(K,),
        in_specs=[
            pl.BlockSpec((NC, N, C), lambda k: (0, 0, 0)),
            pl.BlockSpec((1, C), lambda k: (0, 0)),
            pl.BlockSpec((1, C), lambda k: (0, 0)),
            pl.BlockSpec((1, C, C), lambda k: (k, 0, 0)),
        ],
        out_specs=pl.BlockSpec((1, N, C), lambda k: (k, 0, 0)),
        out_shape=jax.ShapeDtypeStruct((K, N, C), jnp.float32),
        scratch_shapes=[pltpu.VMEM((N, C), jnp.float32)],
    )(p, g, b, W)


def _final_body(p_ref, g_ref, b_ref, x_ref, o_ref):
    o_ref[...] = jnp.maximum(_bn_from(p_ref, g_ref, b_ref) + x_ref[...], 0.0)


def _final(p, g, b, x):
    return pl.pallas_call(
        _final_body,
        grid=(1,),
        in_specs=[
            pl.BlockSpec((NC, N, C), lambda k: (0, 0, 0)),
            pl.BlockSpec((1, C), lambda k: (0, 0)),
            pl.BlockSpec((1, C), lambda k: (0, 0)),
            pl.BlockSpec((N, C), lambda k: (0, 0)),
        ],
        out_specs=pl.BlockSpec((N, C), lambda k: (0, 0)),
        out_shape=jax.ShapeDtypeStruct((N, C), jnp.float32),
    )(p, g, b, x)


def kernel(x, W1, g1, b1, W2, g2, b2, edge_index, edge_offset):
    src = edge_index[0]
    dst = edge_index[1]
    gidx = (edge_offset * N + src).reshape(NW, NCHUNK, CHUNK)
    didx = dst.reshape(NW, NCHUNK, CHUNK)
    g1r = g1.reshape(1, C)
    b1r = b1.reshape(1, C)
    g2r = g2.reshape(1, C)
    b2r = b2.reshape(1, C)

    y1 = _mm(x, W1).reshape(K * N, C)
    p1 = _edge_accum(y1, gidx, didx)
    y2 = _mid(p1, g1r, b1r, W2).reshape(K * N, C)
    p2 = _edge_accum(y2, gidx, didx)
    return _final(p2, g2r, b2r, x)
